# baseline (device time: 22747 ns/iter reference)
import jax
import jax.numpy as jnp
from jax import lax
from jax.experimental import pallas as pl
from jax.experimental.pallas import tpu as pltpu

FWD_SIZES = [8, 8, 16, 32] + [64] * 6 + [24, 8, 8]
MARGIN_SIZES = [48]
Y_SIZES = FWD_SIZES + MARGIN_SIZES
Y_OFFS = [sum(Y_SIZES[:c]) for c in range(len(Y_SIZES))]
N_Y = len(Y_SIZES)
N_FWD = len(FWD_SIZES)


def kernel(x):
    m, n = x.shape
    half = n // 2
    assert sum(Y_SIZES) + sum(FWD_SIZES) == m

    def body(x_ref, out_ref, y_send, y_recv, x_send, x_recv):
        my_x = lax.axis_index("x")
        my_y = lax.axis_index("y")
        peer_y = 1 - my_y
        peer_x = 1 - my_x

        barrier = pltpu.get_barrier_semaphore()
        for dev in ((my_x, peer_y), (peer_x, my_y)):
            pl.semaphore_signal(
                barrier, inc=1, device_id=dev,
                device_id_type=pl.DeviceIdType.MESH,
            )
        pl.semaphore_wait(barrier, 2)

        y_rdmas = []
        for c in range(N_Y):
            sz = Y_SIZES[c]
            r = my_x * (m - 2 * Y_OFFS[c] - sz) + Y_OFFS[c]
            rdma = pltpu.make_async_remote_copy(
                src_ref=x_ref.at[pl.ds(r, sz), pl.ds(peer_y * half, half)],
                dst_ref=out_ref.at[pl.ds(my_y * m + r, sz), :],
                send_sem=y_send.at[c],
                recv_sem=y_recv.at[c],
                device_id=(my_x, peer_y),
                device_id_type=pl.DeviceIdType.MESH,
            )
            rdma.start()
            y_rdmas.append(rdma)

        x_rdmas = []
        for c in range(N_FWD):
            y_rdmas[c].wait_recv()
            sz = Y_SIZES[c]
            r = peer_y * m + my_x * (m - 2 * Y_OFFS[c] - sz) + Y_OFFS[c]
            fwd = pltpu.make_async_remote_copy(
                src_ref=out_ref.at[pl.ds(r, sz), :],
                dst_ref=out_ref.at[pl.ds(r, sz), :],
                send_sem=x_send.at[c],
                recv_sem=x_recv.at[c],
                device_id=(peer_x, my_y),
                device_id_type=pl.DeviceIdType.MESH,
            )
            fwd.start()
            x_rdmas.append(fwd)

        out_ref[pl.ds(my_y * m, m), :] = x_ref[:, pl.ds(my_y * half, half)]

        for c in range(N_FWD, N_Y):
            y_rdmas[c].wait_recv()
        for c in range(N_Y):
            y_rdmas[c].wait_send()
        for c in range(N_FWD):
            x_rdmas[c].wait_recv()
            x_rdmas[c].wait_send()

    out_shape = jax.ShapeDtypeStruct((2 * m, half), x.dtype)
    return pl.pallas_call(
        body,
        out_shape=out_shape,
        in_specs=[pl.BlockSpec(memory_space=pltpu.VMEM)],
        out_specs=pl.BlockSpec(memory_space=pltpu.VMEM),
        scratch_shapes=[
            pltpu.SemaphoreType.DMA((N_Y,)),
            pltpu.SemaphoreType.DMA((N_Y,)),
            pltpu.SemaphoreType.DMA((N_FWD,)),
            pltpu.SemaphoreType.DMA((N_FWD,)),
        ],
        compiler_params=pltpu.CompilerParams(collective_id=0),
    )(x)


# device time: 21927 ns/iter; 1.0374x vs baseline; 1.0374x over previous
import jax
import jax.numpy as jnp
from jax import lax
from jax.experimental import pallas as pl
from jax.experimental.pallas import tpu as pltpu

CH = 32
Y_CHUNKS = 17
FWD_CHUNKS = 15


def kernel(x):
    m, n = x.shape
    half = n // 2
    assert (Y_CHUNKS + FWD_CHUNKS) * CH == m

    def body(x_ref, out_ref, y_send, y_recv, x_send, x_recv):
        my_x = lax.axis_index("x")
        my_y = lax.axis_index("y")
        peer_y = 1 - my_y
        peer_x = 1 - my_x

        base = my_x * (m - CH)
        sign = 1 - 2 * my_x

        barrier = pltpu.get_barrier_semaphore()
        for dev in ((my_x, peer_y), (peer_x, my_y)):
            pl.semaphore_signal(
                barrier, inc=1, device_id=dev,
                device_id_type=pl.DeviceIdType.MESH,
            )
        pl.semaphore_wait(barrier, 2)

        y_rdmas = []
        for c in range(Y_CHUNKS):
            r = base + sign * (c * CH)
            rdma = pltpu.make_async_remote_copy(
                src_ref=x_ref.at[pl.ds(r, CH), pl.ds(peer_y * half, half)],
                dst_ref=out_ref.at[pl.ds(my_y * m + r, CH), :],
                send_sem=y_send.at[c],
                recv_sem=y_recv.at[c],
                device_id=(my_x, peer_y),
                device_id_type=pl.DeviceIdType.MESH,
            )
            rdma.start()
            y_rdmas.append(rdma)

        x_rdmas = []
        for c in range(FWD_CHUNKS):
            y_rdmas[c].wait_recv()
            r = peer_y * m + base + sign * (c * CH)
            fwd = pltpu.make_async_remote_copy(
                src_ref=out_ref.at[pl.ds(r, CH), :],
                dst_ref=out_ref.at[pl.ds(r, CH), :],
                send_sem=x_send.at[c],
                recv_sem=x_recv.at[c],
                device_id=(peer_x, my_y),
                device_id_type=pl.DeviceIdType.MESH,
            )
            fwd.start()
            x_rdmas.append(fwd)

        out_ref[pl.ds(my_y * m, m), :] = x_ref[:, pl.ds(my_y * half, half)]

        for c in range(FWD_CHUNKS, Y_CHUNKS):
            y_rdmas[c].wait_recv()
        for c in range(Y_CHUNKS):
            y_rdmas[c].wait_send()
        for c in range(FWD_CHUNKS):
            x_rdmas[c].wait_recv()
            x_rdmas[c].wait_send()

    out_shape = jax.ShapeDtypeStruct((2 * m, half), x.dtype)
    return pl.pallas_call(
        body,
        out_shape=out_shape,
        in_specs=[pl.BlockSpec(memory_space=pltpu.VMEM)],
        out_specs=pl.BlockSpec(memory_space=pltpu.VMEM),
        scratch_shapes=[
            pltpu.SemaphoreType.DMA((Y_CHUNKS,)),
            pltpu.SemaphoreType.DMA((Y_CHUNKS,)),
            pltpu.SemaphoreType.DMA((FWD_CHUNKS,)),
            pltpu.SemaphoreType.DMA((FWD_CHUNKS,)),
        ],
        compiler_params=pltpu.CompilerParams(collective_id=0),
    )(x)
